# baseline (speedup 1.0000x reference)
import functools

import jax
import jax.numpy as jnp
from jax import lax
from jax.experimental import pallas as pl
from jax.experimental.pallas import tpu as pltpu
from jax.experimental.pallas import tpu_sc as plsc

_NC = 2
_NS = 16
_NW = _NC * _NS
_LANES = 16


def _node_mm_body(x_ref, ws_ref, wr_ref, os_ref, or_ref):
    xv = x_ref[...]
    os_ref[...] = jnp.dot(xv, ws_ref[...], preferred_element_type=jnp.float32)
    or_ref[...] = jnp.dot(xv, wr_ref[...], preferred_element_type=jnp.float32)


def _node_transform(x, wst, wrt):
    n, d = x.shape
    bn = 2000
    return pl.pallas_call(
        _node_mm_body,
        grid=(n // bn,),
        in_specs=[
            pl.BlockSpec((bn, d), lambda i: (i, 0)),
            pl.BlockSpec((d, d), lambda i: (0, 0)),
            pl.BlockSpec((d, d), lambda i: (0, 0)),
        ],
        out_specs=[
            pl.BlockSpec((bn, d), lambda i: (i, 0)),
            pl.BlockSpec((bn, d), lambda i: (i, 0)),
        ],
        out_shape=[
            jax.ShapeDtypeStruct((n, d), jnp.float32),
            jax.ShapeDtypeStruct((n, d), jnp.float32),
        ],
    )(x, wst, wrt)


def _make_gather_sum(e, d, c, npairs):
    epw = e // _NW
    nch = epw // c
    assert epw % c == 0 and c % 8 == 0 and nch >= npairs
    mesh = plsc.VectorSubcoreMesh(core_axis_name="c", subcore_axis_name="s")

    row_buf = [pltpu.VMEM((c, d), jnp.float32) for _ in range(2 * npairs)]
    sems = [pltpu.SemaphoreType.DMA for _ in range(3 * npairs)]

    @functools.partial(
        pl.kernel,
        mesh=mesh,
        out_type=jax.ShapeDtypeStruct((e, d), jnp.float32),
        scratch_types=(
            [pltpu.VMEM((epw,), jnp.int32), pltpu.VMEM((epw,), jnp.int32)]
            + row_buf + sems
        ),
    )
    def gather_sum(xs_hbm, xr_hbm, si_hbm, ri_hbm, out_hbm, si_v, ri_v, *bufs):
        abuf = bufs[0:2 * npairs:2]
        bbuf = bufs[1:2 * npairs:2]
        asem = bufs[2 * npairs:3 * npairs]
        bsem = bufs[3 * npairs:4 * npairs]
        osem = bufs[4 * npairs:5 * npairs]
        wid = lax.axis_index("s") * _NC + lax.axis_index("c")
        base = wid * epw
        pltpu.sync_copy(si_hbm.at[pl.ds(base, epw)], si_v)
        pltpu.sync_copy(ri_hbm.at[pl.ds(base, epw)], ri_v)

        def start_gather(off, p):
            pltpu.async_copy(xs_hbm.at[si_v.at[pl.ds(off, c)]], abuf[p], asem[p])
            pltpu.async_copy(xr_hbm.at[ri_v.at[pl.ds(off, c)]], bbuf[p], bsem[p])

        def wait_gather(p):
            pltpu.make_async_copy(xs_hbm.at[pl.ds(0, c)], abuf[p], asem[p]).wait()
            pltpu.make_async_copy(xr_hbm.at[pl.ds(0, c)], bbuf[p], bsem[p]).wait()

        for p in range(npairs):
            start_gather(p * c, p)

        def do_chunk(o, p):
            wait_gather(p)

            @pl.loop(0, c)
            def _row(i):
                for j in range(d // _LANES):
                    slc = (i, pl.ds(j * _LANES, _LANES))
                    plsc.addupdate(abuf[p].at[slc], bbuf[p][slc])

            pltpu.async_copy(abuf[p], out_hbm.at[pl.ds(base + o, c)], osem[p])

            @pl.when(o + npairs * c < epw)
            def _prefetch():
                pltpu.make_async_copy(
                    abuf[p], out_hbm.at[pl.ds(base, c)], osem[p]).wait()
                start_gather(o + npairs * c, p)

        nmain = npairs * (nch // npairs)

        @pl.loop(0, nmain * c, step=npairs * c)
        def _round(off):
            for p in range(npairs):
                do_chunk(off + p * c, p)

        for t in range(nch - nmain):
            do_chunk((nmain + t) * c, t)

        for p in range(npairs):
            pltpu.make_async_copy(abuf[p], out_hbm.at[pl.ds(base, c)],
                                  osem[p]).wait()

    return gather_sum


def _edge_mm_body(g_ref, ea_ref, we_ref, b_ref, o_ref):
    o_ref[...] = (
        g_ref[...]
        + jnp.dot(ea_ref[...], we_ref[...], preferred_element_type=jnp.float32)
        + b_ref[...]
    )


def _edge_mm_body_alias(v_ref, g_ref, ea_ref, we_ref, b_ref, o_ref):
    del v_ref
    _edge_mm_body(g_ref, ea_ref, we_ref, b_ref, o_ref)


def _edge_combine_chunk(vbuf, g, ea, wet, btot, blk0, e_total, be):
    ec, d = g.shape
    de = ea.shape[1]
    nblk = ec // be
    in_specs = [
        pl.BlockSpec((be, d), lambda i: (i, 0)),
        pl.BlockSpec((be, de), lambda i: (i, 0)),
        pl.BlockSpec((de, d), lambda i: (0, 0)),
        pl.BlockSpec((1, d), lambda i: (0, 0)),
    ]
    out_spec = pl.BlockSpec((be, d), lambda i: (blk0 + i, 0))
    out_shape = jax.ShapeDtypeStruct((e_total, d), jnp.float32)
    if vbuf is None:
        return pl.pallas_call(
            _edge_mm_body,
            grid=(nblk,),
            in_specs=in_specs,
            out_specs=out_spec,
            out_shape=out_shape,
        )(g, ea, wet, btot)
    return pl.pallas_call(
        _edge_mm_body_alias,
        grid=(nblk,),
        in_specs=[pl.BlockSpec(memory_space=pl.ANY)] + in_specs,
        out_specs=out_spec,
        out_shape=out_shape,
        input_output_aliases={0: 0},
    )(vbuf, g, ea, wet, btot)


def kernel(x, edge_index, edge_attr, Ws, bs, Wr, br, We, be):
    e = edge_index.shape[1]
    d = x.shape[1]
    senders = edge_index[0]
    receivers = edge_index[1]
    xs, xr = _node_transform(x, Ws.T, Wr.T)
    btot = (bs + br + be).reshape(1, d)
    wet = We.T

    nchunks = 2
    ec = e // nchunks
    be_blk = 3200
    sc_gather = _make_gather_sum(ec, d, 40, 4)
    gs = [sc_gather(xs, xr, senders[k * ec:(k + 1) * ec],
                    receivers[k * ec:(k + 1) * ec]) for k in range(nchunks)]
    v = None
    for k in range(nchunks):
        v = _edge_combine_chunk(v, gs[k], edge_attr[k * ec:(k + 1) * ec],
                                wet, btot, k * (ec // be_blk), e, be_blk)
    return v


# speedup vs baseline: 1.0203x; 1.0203x over previous
import functools

import jax
import jax.numpy as jnp
from jax import lax
from jax.experimental import pallas as pl
from jax.experimental.pallas import tpu as pltpu
from jax.experimental.pallas import tpu_sc as plsc

_NC = 2
_NS = 16
_NW = _NC * _NS
_LANES = 16


_DNT = (((1,), (1,)), ((), ()))


def _node_mm_body(x_ref, ws_ref, wr_ref, os_ref, or_ref):
    xv = x_ref[...]
    os_ref[...] = lax.dot_general(xv, ws_ref[...], _DNT,
                                  preferred_element_type=jnp.float32)
    or_ref[...] = lax.dot_general(xv, wr_ref[...], _DNT,
                                  preferred_element_type=jnp.float32)


def _node_transform(x, ws, wr):
    n, d = x.shape
    bn = 2000
    return pl.pallas_call(
        _node_mm_body,
        grid=(n // bn,),
        in_specs=[
            pl.BlockSpec((bn, d), lambda i: (i, 0)),
            pl.BlockSpec((d, d), lambda i: (0, 0)),
            pl.BlockSpec((d, d), lambda i: (0, 0)),
        ],
        out_specs=[
            pl.BlockSpec((bn, d), lambda i: (i, 0)),
            pl.BlockSpec((bn, d), lambda i: (i, 0)),
        ],
        out_shape=[
            jax.ShapeDtypeStruct((n, d), jnp.float32),
            jax.ShapeDtypeStruct((n, d), jnp.float32),
        ],
    )(x, ws, wr)


def _make_gather_sum(e, d, c, npairs):
    epw = e // _NW
    nch = epw // c
    assert epw % c == 0 and c % 8 == 0 and nch >= npairs
    mesh = plsc.VectorSubcoreMesh(core_axis_name="c", subcore_axis_name="s")

    row_buf = [pltpu.VMEM((c, d), jnp.float32) for _ in range(2 * npairs)]
    sems = [pltpu.SemaphoreType.DMA for _ in range(3 * npairs)]

    @functools.partial(
        pl.kernel,
        mesh=mesh,
        out_type=jax.ShapeDtypeStruct((e, d), jnp.float32),
        scratch_types=(
            [pltpu.VMEM((epw,), jnp.int32), pltpu.VMEM((epw,), jnp.int32)]
            + row_buf + sems
        ),
    )
    def gather_sum(xs_hbm, xr_hbm, si_hbm, ri_hbm, out_hbm, si_v, ri_v, *bufs):
        abuf = bufs[0:2 * npairs:2]
        bbuf = bufs[1:2 * npairs:2]
        asem = bufs[2 * npairs:3 * npairs]
        bsem = bufs[3 * npairs:4 * npairs]
        osem = bufs[4 * npairs:5 * npairs]
        wid = lax.axis_index("s") * _NC + lax.axis_index("c")
        base = wid * epw
        pltpu.sync_copy(si_hbm.at[pl.ds(base, epw)], si_v)
        pltpu.sync_copy(ri_hbm.at[pl.ds(base, epw)], ri_v)

        def start_gather(off, p):
            pltpu.async_copy(xs_hbm.at[si_v.at[pl.ds(off, c)]], abuf[p], asem[p])
            pltpu.async_copy(xr_hbm.at[ri_v.at[pl.ds(off, c)]], bbuf[p], bsem[p])

        def wait_gather(p):
            pltpu.make_async_copy(xs_hbm.at[pl.ds(0, c)], abuf[p], asem[p]).wait()
            pltpu.make_async_copy(xr_hbm.at[pl.ds(0, c)], bbuf[p], bsem[p]).wait()

        for p in range(npairs):
            start_gather(p * c, p)

        def do_chunk(o, p):
            wait_gather(p)

            @pl.loop(0, c)
            def _row(i):
                for j in range(d // _LANES):
                    slc = (i, pl.ds(j * _LANES, _LANES))
                    plsc.addupdate(abuf[p].at[slc], bbuf[p][slc])

            pltpu.async_copy(abuf[p], out_hbm.at[pl.ds(base + o, c)], osem[p])

            @pl.when(o + npairs * c < epw)
            def _prefetch():
                pltpu.make_async_copy(
                    abuf[p], out_hbm.at[pl.ds(base, c)], osem[p]).wait()
                start_gather(o + npairs * c, p)

        nmain = npairs * (nch // npairs)

        @pl.loop(0, nmain * c, step=npairs * c)
        def _round(off):
            for p in range(npairs):
                do_chunk(off + p * c, p)

        for t in range(nch - nmain):
            do_chunk((nmain + t) * c, t)

        for p in range(npairs):
            pltpu.make_async_copy(abuf[p], out_hbm.at[pl.ds(base, c)],
                                  osem[p]).wait()

    return gather_sum


def _edge_mm_body(g_ref, ea_ref, we_ref, b_ref, o_ref):
    o_ref[...] = (
        g_ref[...]
        + lax.dot_general(ea_ref[...], we_ref[...], _DNT,
                          preferred_element_type=jnp.float32)
        + b_ref[...]
    )


def _edge_mm_body_alias(v_ref, g_ref, ea_ref, we_ref, b_ref, o_ref):
    del v_ref
    _edge_mm_body(g_ref, ea_ref, we_ref, b_ref, o_ref)


def _edge_combine_chunk(vbuf, g, ea, we, btot, blk0, e_total, be):
    ec, d = g.shape
    de = ea.shape[1]
    nblk = ec // be
    in_specs = [
        pl.BlockSpec((be, d), lambda i: (i, 0)),
        pl.BlockSpec((be, de), lambda i: (i, 0)),
        pl.BlockSpec((d, de), lambda i: (0, 0)),
        pl.BlockSpec((1, d), lambda i: (0, 0)),
    ]
    out_spec = pl.BlockSpec((be, d), lambda i: (blk0 + i, 0))
    out_shape = jax.ShapeDtypeStruct((e_total, d), jnp.float32)
    if vbuf is None:
        return pl.pallas_call(
            _edge_mm_body,
            grid=(nblk,),
            in_specs=in_specs,
            out_specs=out_spec,
            out_shape=out_shape,
        )(g, ea, we, btot)
    return pl.pallas_call(
        _edge_mm_body_alias,
        grid=(nblk,),
        in_specs=[pl.BlockSpec(memory_space=pl.ANY)] + in_specs,
        out_specs=out_spec,
        out_shape=out_shape,
        input_output_aliases={0: 0},
    )(vbuf, g, ea, we, btot)


def kernel(x, edge_index, edge_attr, Ws, bs, Wr, br, We, be):
    e = edge_index.shape[1]
    d = x.shape[1]
    xs, xr = _node_transform(x, Ws, Wr)
    gsum = _make_gather_sum(e, d, 80, 4)(xs, xr, edge_index[0], edge_index[1])
    btot = (bs + br + be).reshape(1, d)
    return _edge_combine_chunk(None, gsum, edge_attr, We, btot, 0, e, 3200)


# speedup vs baseline: 1.0514x; 1.0305x over previous
import functools

import jax
import jax.numpy as jnp
from jax import lax
from jax.experimental import pallas as pl
from jax.experimental.pallas import tpu as pltpu
from jax.experimental.pallas import tpu_sc as plsc

_NC = 2
_NS = 16
_NW = _NC * _NS
_LANES = 16


_DNT = (((1,), (1,)), ((), ()))


def _node_mm_body(x_ref, ws_ref, wr_ref, os_ref, or_ref):
    xv = x_ref[...]
    os_ref[...] = lax.dot_general(xv, ws_ref[...], _DNT,
                                  preferred_element_type=jnp.float32)
    or_ref[...] = lax.dot_general(xv, wr_ref[...], _DNT,
                                  preferred_element_type=jnp.float32)


def _node_transform(x, ws, wr):
    n, d = x.shape
    bn = 2000
    return pl.pallas_call(
        _node_mm_body,
        grid=(n // bn,),
        in_specs=[
            pl.BlockSpec((bn, d), lambda i: (i, 0)),
            pl.BlockSpec((d, d), lambda i: (0, 0)),
            pl.BlockSpec((d, d), lambda i: (0, 0)),
        ],
        out_specs=[
            pl.BlockSpec((bn, d), lambda i: (i, 0)),
            pl.BlockSpec((bn, d), lambda i: (i, 0)),
        ],
        out_shape=[
            jax.ShapeDtypeStruct((n, d), jnp.float32),
            jax.ShapeDtypeStruct((n, d), jnp.float32),
        ],
    )(x, ws, wr)


def _make_gather_sum(e, d, c, npairs):
    epw = e // _NW
    nch = epw // c
    assert epw % c == 0 and c % 8 == 0 and nch >= npairs
    mesh = plsc.VectorSubcoreMesh(core_axis_name="c", subcore_axis_name="s")

    row_buf = [pltpu.VMEM((c, d), jnp.float32) for _ in range(2 * npairs)]
    sems = [pltpu.SemaphoreType.DMA for _ in range(3 * npairs)]

    @functools.partial(
        pl.kernel,
        mesh=mesh,
        out_type=jax.ShapeDtypeStruct((e, d), jnp.float32),
        scratch_types=(
            [pltpu.VMEM((epw,), jnp.int32), pltpu.VMEM((epw,), jnp.int32)]
            + row_buf + sems
        ),
    )
    def gather_sum(xs_hbm, xr_hbm, si_hbm, ri_hbm, out_hbm, si_v, ri_v, *bufs):
        abuf = bufs[0:2 * npairs:2]
        bbuf = bufs[1:2 * npairs:2]
        asem = bufs[2 * npairs:3 * npairs]
        bsem = bufs[3 * npairs:4 * npairs]
        osem = bufs[4 * npairs:5 * npairs]
        wid = lax.axis_index("s") * _NC + lax.axis_index("c")
        base = wid * epw
        pltpu.sync_copy(si_hbm.at[pl.ds(base, epw)], si_v)
        pltpu.sync_copy(ri_hbm.at[pl.ds(base, epw)], ri_v)

        def start_gather(off, p):
            pltpu.async_copy(xs_hbm.at[si_v.at[pl.ds(off, c)]], abuf[p], asem[p])
            pltpu.async_copy(xr_hbm.at[ri_v.at[pl.ds(off, c)]], bbuf[p], bsem[p])

        def wait_gather(p):
            pltpu.make_async_copy(xs_hbm.at[pl.ds(0, c)], abuf[p], asem[p]).wait()
            pltpu.make_async_copy(xr_hbm.at[pl.ds(0, c)], bbuf[p], bsem[p]).wait()

        for p in range(npairs):
            start_gather(p * c, p)

        def do_chunk(o, p):
            wait_gather(p)

            @pl.loop(0, c)
            def _row(i):
                for j in range(d // _LANES):
                    slc = (i, pl.ds(j * _LANES, _LANES))
                    plsc.addupdate(abuf[p].at[slc], bbuf[p][slc])

            pltpu.async_copy(abuf[p], out_hbm.at[pl.ds(base + o, c)], osem[p])

            @pl.when(o + npairs * c < epw)
            def _prefetch():
                pltpu.make_async_copy(
                    abuf[p], out_hbm.at[pl.ds(base, c)], osem[p]).wait()
                start_gather(o + npairs * c, p)

        nmain = npairs * (nch // npairs)

        @pl.loop(0, nmain * c, step=npairs * c)
        def _round(off):
            for p in range(npairs):
                do_chunk(off + p * c, p)

        for t in range(nch - nmain):
            do_chunk((nmain + t) * c, t)

        for p in range(npairs):
            pltpu.make_async_copy(abuf[p], out_hbm.at[pl.ds(base, c)],
                                  osem[p]).wait()

    return gather_sum


def _edge_mm_body(g_ref, ea_ref, we_ref, b_ref, o_ref):
    o_ref[...] = (
        g_ref[...]
        + lax.dot_general(ea_ref[...], we_ref[...], _DNT,
                          preferred_element_type=jnp.float32)
        + b_ref[...]
    )


def _edge_mm_body_alias(v_ref, g_ref, ea_ref, we_ref, b_ref, o_ref):
    del v_ref
    _edge_mm_body(g_ref, ea_ref, we_ref, b_ref, o_ref)


def _edge_combine_chunk(vbuf, g, ea, we, btot, blk0, e_total, be):
    ec, d = g.shape
    de = ea.shape[1]
    nblk = ec // be
    in_specs = [
        pl.BlockSpec((be, d), lambda i: (i, 0)),
        pl.BlockSpec((be, de), lambda i: (i, 0)),
        pl.BlockSpec((d, de), lambda i: (0, 0)),
        pl.BlockSpec((1, d), lambda i: (0, 0)),
    ]
    out_spec = pl.BlockSpec((be, d), lambda i: (blk0 + i, 0))
    out_shape = jax.ShapeDtypeStruct((e_total, d), jnp.float32)
    if vbuf is None:
        return pl.pallas_call(
            _edge_mm_body,
            grid=(nblk,),
            in_specs=in_specs,
            out_specs=out_spec,
            out_shape=out_shape,
        )(g, ea, we, btot)
    return pl.pallas_call(
        _edge_mm_body_alias,
        grid=(nblk,),
        in_specs=[pl.BlockSpec(memory_space=pl.ANY)] + in_specs,
        out_specs=out_spec,
        out_shape=out_shape,
        input_output_aliases={0: 0},
    )(vbuf, g, ea, we, btot)


def kernel(x, edge_index, edge_attr, Ws, bs, Wr, br, We, be):
    e = edge_index.shape[1]
    d = x.shape[1]
    xs, xr = _node_transform(x, Ws, Wr)
    gsum = _make_gather_sum(e, d, 80, 4)(xs, xr, edge_index[0], edge_index[1])
    btot = (bs + br + be).reshape(1, d)
    return _edge_combine_chunk(None, gsum, edge_attr, We, btot, 0, e, 6400)
